# Initial kernel scaffold; baseline (speedup 1.0000x reference)
import jax
import jax.numpy as jnp
from jax.experimental import pallas as pl


def kernel(indices, table):
    raise NotImplementedError("write your pallas kernel here")


# speedup vs baseline: 1.1106x; 1.1106x over previous
import functools

import jax
import jax.numpy as jnp
from jax import lax
from jax.experimental import pallas as pl
from jax.experimental.pallas import tpu as pltpu
from jax.experimental.pallas import tpu_sc as plsc

_info = plsc.get_sparse_core_info()
_NC, _NS = _info.num_cores, _info.num_subcores
_NW = _NC * _NS

_CHUNK = 128
_K = 20


def _sc_gather(idx_grp, table, n_rows, dim):
    n_chunks = idx_grp.shape[1]
    per_w = n_chunks * _CHUNK
    n_batches = n_chunks // _K
    mesh = plsc.VectorSubcoreMesh(core_axis_name="c", subcore_axis_name="s")

    @functools.partial(
        pl.kernel,
        mesh=mesh,
        compiler_params=pltpu.CompilerParams(use_tc_tiling_on_sc=False),
        out_type=jax.ShapeDtypeStruct((n_rows, dim), jnp.float32),
        scratch_types=[
            pltpu.VMEM((n_chunks, _CHUNK), jnp.int32),
            pltpu.VMEM((_K * _CHUNK, dim), jnp.float32),
            pltpu.SemaphoreType.DMA,
        ],
    )
    def k(idx_hbm, table_hbm, out_hbm, idx_v, rows_v, sem):
        wid = lax.axis_index("s") * _NC + lax.axis_index("c")
        base = wid * per_w
        pltpu.sync_copy(idx_hbm.at[wid], idx_v)

        def batch_body(b, carry):
            waits = []
            for j in range(_K):
                waits.append(pltpu.async_copy(
                    table_hbm.at[idx_v.at[b * _K + j]],
                    rows_v.at[pl.ds(j * _CHUNK, _CHUNK)],
                    sem,
                ))
            for w in waits:
                w.wait()
            pltpu.sync_copy(
                rows_v,
                out_hbm.at[pl.ds(base + b * (_K * _CHUNK), _K * _CHUNK)],
            )
            return carry

        lax.fori_loop(0, n_batches, batch_body, 0)

    return k(idx_grp, table)


def kernel(indices, table):
    batch, hist = indices.shape
    vocab, dim = table.shape
    n_rows = batch * hist
    assert n_rows % (_NW * _CHUNK * _K) == 0
    idx_flat = indices.astype(jnp.int32).reshape(_NW, n_rows // (_NW * _CHUNK), _CHUNK)
    out = _sc_gather(idx_flat, table, n_rows, dim)
    return out.reshape(batch, hist, dim)


# speedup vs baseline: 1.5874x; 1.4293x over previous
import functools

import jax
import jax.numpy as jnp
from jax import lax
from jax.experimental import pallas as pl
from jax.experimental.pallas import tpu as pltpu
from jax.experimental.pallas import tpu_sc as plsc

_info = plsc.get_sparse_core_info()
_NC, _NS = _info.num_cores, _info.num_subcores
_NW = _NC * _NS

_CHUNK = 128


def _sc_gather(idx_t, table):
    hist, batch = idx_t.shape
    _, dim = table.shape
    cpw = batch // (_NW * _CHUNK)
    bpw = cpw * _CHUNK
    mesh = plsc.VectorSubcoreMesh(core_axis_name="c", subcore_axis_name="s")

    @functools.partial(
        pl.kernel,
        mesh=mesh,
        compiler_params=pltpu.CompilerParams(use_tc_tiling_on_sc=False),
        out_type=jax.ShapeDtypeStruct((batch, hist, dim), jnp.float32),
        scratch_types=[
            pltpu.VMEM((hist, bpw), jnp.int32),
            pltpu.VMEM((_CHUNK, dim), jnp.float32),
            pltpu.SemaphoreType.DMA,
        ],
    )
    def k(idx_hbm, table_hbm, out_hbm, idx_v, rows_v, sem):
        wid = lax.axis_index("s") * _NC + lax.axis_index("c")
        b_base = wid * bpw
        pltpu.sync_copy(idx_hbm.at[:, pl.ds(b_base, bpw)], idx_v)

        def unit_body(u, carry):
            h = u // cpw
            j = u % cpw
            pltpu.async_copy(
                table_hbm.at[idx_v.at[h, pl.ds(j * _CHUNK, _CHUNK)]],
                rows_v,
                sem,
            ).wait()
            pltpu.sync_copy(
                rows_v,
                out_hbm.at[pl.ds(b_base + j * _CHUNK, _CHUNK), h],
            )
            return carry

        lax.fori_loop(0, hist * cpw, unit_body, 0)

    return k(idx_t, table)


def kernel(indices, table):
    batch, hist = indices.shape
    assert batch % (_NW * _CHUNK) == 0
    return _sc_gather(indices.astype(jnp.int32).T, table)
